# baseline (speedup 1.0000x reference)
import functools

import jax
import jax.numpy as jnp
from jax import lax
from jax.experimental import pallas as pl
from jax.experimental.pallas import tpu as pltpu
from jax.experimental.pallas import tpu_sc as plsc

B, Q, L, D, HW = 4, 1, 16, 300, 224
N = HW * HW
IDX = 32
BG = 16

P = 128
CPB = N // P
NW = 32
TPB = NW // B
CPT = CPB // TPB


def _prologue_body(emb_ref, q_ref, wo_ref, bo_ref, wq_ref, bq_ref,
                   bbox_ref, len_ref,
                   embT_ref, embx_ref, qout_ref, s_ref, cnt_ref):
    e2 = emb_ref[...].reshape(B * L, D)
    h = lax.dot_general(e2, wo_ref[...], (((1,), (1,)), ((), ())),
                        preferred_element_type=jnp.float32) + bo_ref[...]
    h = jnp.maximum(h, 0.0)
    hn = jnp.sqrt(jnp.sum(h * h, axis=1, keepdims=True))
    h = h / jnp.maximum(hn, 1e-12)

    q2 = q_ref[...].reshape(B * Q, D)
    qh = lax.dot_general(q2, wq_ref[...], (((1,), (1,)), ((), ())),
                         preferred_element_type=jnp.float32) + bq_ref[...]
    qh = jnp.maximum(qh, 0.0)
    qn = jnp.sqrt(jnp.sum(qh * qh, axis=1, keepdims=True))
    qh = qh / jnp.maximum(qn, 1e-12)
    qout_ref[...] = qh.reshape(B, Q, D)

    ext = jnp.concatenate(
        [h.reshape(B, L, D), jnp.zeros((B, IDX - L, D), jnp.float32)], axis=1)
    embx_ref[...] = ext
    embT_ref[...] = jnp.transpose(ext, (0, 2, 1))
    s_ref[...] = lax.dot_general(qh.reshape(B, Q, D), ext,
                                 (((2,), (2,)), ((0,), (0,))),
                                 preferred_element_type=jnp.float32)

    w8 = N // 8
    pix = (lax.broadcasted_iota(jnp.int32, (8, w8), 0) * w8
           + lax.broadcasted_iota(jnp.int32, (8, w8), 1))
    r = pix // HW
    c = pix % HW
    for b in range(B):
        idx = jnp.full((8, w8), BG, jnp.int32)
        elen = len_ref[b]
        for i in range(L):
            x = bbox_ref[b, i, 0]
            y = bbox_ref[b, i, 1]
            x2 = bbox_ref[b, i, 2]
            y2 = bbox_ref[b, i, 3]
            cov = (r >= y) & (r < y2) & (c >= x) & (c < x2) & (i < elen)
            idx = jnp.where(cov, i, idx)
        for j in range(IDX):
            if j <= BG:
                cnt_ref[b, 0, j] = jnp.sum((idx == j).astype(jnp.float32))
            else:
                cnt_ref[b, 0, j] = 0.0


def _sc_materialize(tab_hbm, bb_hbm, elen_hbm, wg_hbm,
                    tab_v, bb_v, elen_v, sems):
    pl.run_scoped(
        functools.partial(_sc_materialize_inner, tab_hbm, bb_hbm, elen_hbm,
                          wg_hbm, tab_v, bb_v, elen_v, sems),
        pltpu.VMEM((2, D, P), jnp.float32))


def _sc_materialize_inner(tab_hbm, bb_hbm, elen_hbm, wg_hbm,
                          tab_v, bb_v, elen_v, sems, slab_v):
    cidx = lax.axis_index("c")
    sidx = lax.axis_index("s")
    wid = sidx * 2 + cidx
    b = wid // TPB
    chunk0 = (wid % TPB) * CPT

    pltpu.sync_copy(tab_hbm, tab_v)
    pltpu.sync_copy(bb_hbm, bb_v)
    pltpu.sync_copy(elen_hbm, elen_v)

    lanes = lax.iota(jnp.int32, 16)
    elen_vec = plsc.load_gather(elen_v, [jnp.full((16,), b * 16, jnp.int32) + lanes])

    def splat(x):
        return jnp.full((16,), x, jnp.int32)

    def mk_copy(buf, cp):
        return pltpu.make_async_copy(
            slab_v.at[buf],
            wg_hbm.at[b, :, pl.ds(cp, P)],
            sems.at[buf])

    def chunk_work(ci, buf):
        cp = pl.multiple_of((chunk0 + ci) * P, P)
        rs, cs = [], []
        for v in range(8):
            p = splat(cp + v * 16) + lanes
            r = ((p >> 5) * 9363) >> 16
            rs.append(r)
            cs.append(p - r * HW)

        def box_step(i, idxs):
            iv = splat(i)
            xi = plsc.load_gather(bb_v, [splat(b * 64 + 0) + iv])
            yi = plsc.load_gather(bb_v, [splat(b * 64 + 16) + iv])
            x2i = plsc.load_gather(bb_v, [splat(b * 64 + 32) + iv])
            y2i = plsc.load_gather(bb_v, [splat(b * 64 + 48) + iv])
            ilt = iv < elen_vec
            out = []
            for v in range(8):
                cov = ((rs[v] >= yi) & (rs[v] < y2i)
                       & (cs[v] >= xi) & (cs[v] < x2i) & ilt)
                out.append(jnp.where(cov, i, idxs[v]))
            return tuple(out)

        idxs = lax.fori_loop(0, L, box_step,
                             tuple(splat(BG) for _ in range(8)))
        tb = splat(b * (D * IDX))
        idxs = tuple(ix + tb for ix in idxs)

        @pl.when(ci >= 2)
        def _():
            mk_copy(buf, cp).wait()

        def d_step(d, addrs):
            for v in range(8):
                vals = plsc.load_gather(tab_v, [addrs[v]])
                slab_v[buf, d, pl.ds(v * 16, 16)] = vals
            return tuple(a + 32 for a in addrs)

        lax.fori_loop(0, D, d_step, idxs)
        mk_copy(buf, cp).start()

    def pair(j, carry):
        chunk_work(2 * j, 0)
        chunk_work(2 * j + 1, 1)
        return carry

    lax.fori_loop(0, CPT // 2, pair, 0)
    chunk_work(CPT - 1, 0)

    last = pl.multiple_of((chunk0 + CPT - 1) * P, P)
    mk_copy(0, last).wait()
    mk_copy(1, last).wait()


_sc_materialize_call = functools.partial(
    pl.kernel,
    out_type=jax.ShapeDtypeStruct((B, D, N), jnp.float32),
    mesh=plsc.VectorSubcoreMesh(core_axis_name="c", subcore_axis_name="s"),
    scratch_types=[
        pltpu.VMEM((B * D * IDX,), jnp.float32),
        pltpu.VMEM((B * 4 * L,), jnp.int32),
        pltpu.VMEM((B * 16,), jnp.int32),
        pltpu.SemaphoreType.DMA((2,)),
    ],
    compiler_params=pltpu.CompilerParams(needs_layout_passes=False),
)(_sc_materialize)


def _final_body(cnt_ref, s_ref, embx_ref, out_ref):
    c = cnt_ref[...].reshape(B, IDX)
    sv = s_ref[...].reshape(B, IDX)
    active = c > 0.0
    m = jnp.max(jnp.where(active, sv, -1e30), axis=1, keepdims=True)
    e = jnp.where(active, jnp.exp(sv - m), 0.0)
    w = c * e
    z = jnp.sum(w, axis=1, keepdims=True)
    coef = w / z
    wa = lax.dot_general(coef, embx_ref[...], (((1,), (1,)), ((0,), (0,))),
                         preferred_element_type=jnp.float32)
    nrm = jnp.sqrt(jnp.sum(wa * wa, axis=0, keepdims=True))
    out_ref[...] = wa / jnp.maximum(nrm, 1e-12)


def kernel(image, question, question_len, embeddings, bboxes, emb_lengths,
           W_ocr, b_ocr, W_q, b_q):
    del image, question_len

    embT, embx, qout, s, counts = pl.pallas_call(
        _prologue_body,
        in_specs=[
            pl.BlockSpec(memory_space=pltpu.VMEM),
            pl.BlockSpec(memory_space=pltpu.VMEM),
            pl.BlockSpec(memory_space=pltpu.VMEM),
            pl.BlockSpec(memory_space=pltpu.VMEM),
            pl.BlockSpec(memory_space=pltpu.VMEM),
            pl.BlockSpec(memory_space=pltpu.VMEM),
            pl.BlockSpec(memory_space=pltpu.SMEM),
            pl.BlockSpec(memory_space=pltpu.SMEM),
        ],
        out_specs=[
            pl.BlockSpec(memory_space=pltpu.VMEM),
            pl.BlockSpec(memory_space=pltpu.VMEM),
            pl.BlockSpec(memory_space=pltpu.VMEM),
            pl.BlockSpec(memory_space=pltpu.VMEM),
            pl.BlockSpec(memory_space=pltpu.SMEM),
        ],
        out_shape=(
            jax.ShapeDtypeStruct((B, D, IDX), jnp.float32),
            jax.ShapeDtypeStruct((B, IDX, D), jnp.float32),
            jax.ShapeDtypeStruct((B, Q, D), jnp.float32),
            jax.ShapeDtypeStruct((B, Q, IDX), jnp.float32),
            jax.ShapeDtypeStruct((B, 1, IDX), jnp.float32),
        ),
    )(embeddings, question, W_ocr, b_ocr.reshape(1, D), W_q,
      b_q.reshape(1, D), bboxes, emb_lengths)

    tabflat = embT.reshape(B * D * IDX)
    bb1d = jnp.transpose(bboxes, (0, 2, 1)).reshape(B * 4 * L)
    elen1d = jnp.broadcast_to(emb_lengths[:, None], (B, 16)).reshape(B * 16)

    wordgrid = _sc_materialize_call(tabflat, bb1d, elen1d)

    wavg = pl.pallas_call(
        _final_body,
        out_shape=jax.ShapeDtypeStruct((B, D), jnp.float32),
    )(counts, s, embx)

    return (wavg, qout, wordgrid)


# speedup vs baseline: 6.1490x; 6.1490x over previous
import functools

import jax
import jax.numpy as jnp
from jax import lax
from jax.experimental import pallas as pl
from jax.experimental.pallas import tpu as pltpu

B, Q, L, D, HW = 4, 1, 16, 300, 224
N = HW * HW
IDX = 32
BG = 16
NT = 8
NBUF = 2
NBLK = N // NT


def _prologue_body(emb_ref, q_ref, wo_ref, bo_ref, wq_ref, bq_ref,
                   embT_ref, embx_ref, qout_ref, s_ref):
    e2 = emb_ref[...].reshape(B * L, D)
    h = lax.dot_general(e2, wo_ref[...], (((1,), (1,)), ((), ())),
                        preferred_element_type=jnp.float32) + bo_ref[...]
    h = jnp.maximum(h, 0.0)
    hn = jnp.sqrt(jnp.sum(h * h, axis=1, keepdims=True))
    h = h / jnp.maximum(hn, 1e-12)

    q2 = q_ref[...].reshape(B * Q, D)
    qh = lax.dot_general(q2, wq_ref[...], (((1,), (1,)), ((), ())),
                         preferred_element_type=jnp.float32) + bq_ref[...]
    qh = jnp.maximum(qh, 0.0)
    qn = jnp.sqrt(jnp.sum(qh * qh, axis=1, keepdims=True))
    qh = qh / jnp.maximum(qn, 1e-12)
    qout_ref[...] = qh.reshape(B, Q, D)

    ext = jnp.concatenate(
        [h.reshape(B, L, D), jnp.zeros((B, IDX - L, D), jnp.float32)], axis=1)
    embx_ref[...] = ext
    embT_ref[...] = jnp.transpose(ext, (0, 2, 1))
    s_ref[...] = lax.dot_general(qh.reshape(B, Q, D), ext,
                                 (((2,), (2,)), ((0,), (0,))),
                                 preferred_element_type=jnp.float32)


def _grid_body(bbox_ref, len_ref, embT_ref, wg_ref, cnt_ref,
               oh_ref, slab_ref, sem_ref):
    b = pl.program_id(0)
    p = lax.broadcasted_iota(jnp.int32, (1, N), 1)
    r = p // HW
    c = p % HW
    idx = jnp.full((1, N), BG, jnp.int32)
    elen = len_ref[b]
    for i in range(L):
        x = bbox_ref[b, i, 0]
        y = bbox_ref[b, i, 1]
        x2 = bbox_ref[b, i, 2]
        y2 = bbox_ref[b, i, 3]
        cov = (r >= y) & (r < y2) & (c >= x) & (c < x2) & (i < elen)
        idx = jnp.where(cov, i, idx)
    oh = (lax.broadcasted_iota(jnp.int32, (IDX, N), 0) == idx
          ).astype(jnp.float32)
    oh_ref[...] = oh
    cnt_ref[0] = lax.dot_general(jnp.ones((1, N), jnp.float32), oh,
                                 (((1,), (1,)), ((), ())),
                                 preferred_element_type=jnp.float32)

    def _copy(buf, nt):
        return pltpu.make_async_copy(
            slab_ref.at[buf],
            wg_ref.at[:, b, pl.ds(nt * NBLK, NBLK)],
            sem_ref.at[buf])

    for nt in range(NT):
        buf = nt % NBUF
        if nt >= NBUF:
            _copy(buf, nt - NBUF).wait()
        else:
            @pl.when(b > 0)
            def _():
                _copy(buf, nt).wait()
        slab_ref[buf] = lax.dot_general(
            embT_ref[0], oh_ref[:, nt * NBLK:(nt + 1) * NBLK],
            (((1,), (0,)), ((), ())), preferred_element_type=jnp.float32)
        _copy(buf, nt).start()

    @pl.when(b == B - 1)
    def _():
        for k in range(NBUF):
            _copy(k, NT - NBUF + k).wait()


def _final_body(cnt_ref, s_ref, embx_ref, out_ref):
    c = cnt_ref[...].reshape(B, IDX)
    sv = s_ref[...].reshape(B, IDX)
    active = c > 0.0
    m = jnp.max(jnp.where(active, sv, -1e30), axis=1, keepdims=True)
    e = jnp.where(active, jnp.exp(sv - m), 0.0)
    w = c * e
    z = jnp.sum(w, axis=1, keepdims=True)
    coef = w / z
    wa = lax.dot_general(coef, embx_ref[...], (((1,), (1,)), ((0,), (0,))),
                         preferred_element_type=jnp.float32)
    nrm = jnp.sqrt(jnp.sum(wa * wa, axis=0, keepdims=True))
    out_ref[...] = wa / jnp.maximum(nrm, 1e-12)


def kernel(image, question, question_len, embeddings, bboxes, emb_lengths,
           W_ocr, b_ocr, W_q, b_q):
    del image, question_len

    embT, embx, qout, s = pl.pallas_call(
        _prologue_body,
        out_shape=(
            jax.ShapeDtypeStruct((B, D, IDX), jnp.float32),
            jax.ShapeDtypeStruct((B, IDX, D), jnp.float32),
            jax.ShapeDtypeStruct((B, Q, D), jnp.float32),
            jax.ShapeDtypeStruct((B, Q, IDX), jnp.float32),
        ),
    )(embeddings, question, W_ocr, b_ocr.reshape(1, D), W_q, b_q.reshape(1, D))

    wgT, counts = pl.pallas_call(
        _grid_body,
        grid=(B,),
        in_specs=[
            pl.BlockSpec(memory_space=pltpu.SMEM),
            pl.BlockSpec(memory_space=pltpu.SMEM),
            pl.BlockSpec((1, D, IDX), lambda b: (b, 0, 0)),
        ],
        out_specs=[
            pl.BlockSpec(memory_space=pl.ANY),
            pl.BlockSpec((1, 1, IDX), lambda b: (b, 0, 0)),
        ],
        out_shape=(
            jax.ShapeDtypeStruct((D, B, N), jnp.float32),
            jax.ShapeDtypeStruct((B, 1, IDX), jnp.float32),
        ),
        scratch_shapes=[
            pltpu.VMEM((IDX, N), jnp.float32),
            pltpu.VMEM((NBUF, D, NBLK), jnp.float32),
            pltpu.SemaphoreType.DMA((NBUF,)),
        ],
    )(bboxes, emb_lengths, embT)
    wordgrid = jnp.transpose(wgT, (1, 0, 2))

    wavg = pl.pallas_call(
        _final_body,
        out_shape=jax.ShapeDtypeStruct((B, D), jnp.float32),
    )(counts, s, embx)

    return (wavg, qout, wordgrid)
